# baseline (speedup 1.0000x reference)
import functools

import jax
import jax.numpy as jnp
from jax import lax
from jax.experimental import pallas as pl
from jax.experimental.pallas import tpu as pltpu
from jax.experimental.pallas import tpu_sc as plsc

_D = 1024
_ROWS = 512
_NC = 2
_NS = 16
_NW = _NC * _NS
_CHUNK = 64


def _sc_phase(mask_i32, emb_weight, n_rows):
    rpw = n_rows // _NW
    n_chunks = rpw // _CHUNK
    mesh = plsc.VectorSubcoreMesh(core_axis_name="c", subcore_axis_name="s")

    @functools.partial(
        pl.kernel,
        mesh=mesh,
        out_type=jax.ShapeDtypeStruct((n_rows, 2 * _D), jnp.float32),
        scratch_types=[
            pltpu.VMEM((rpw,), jnp.int32),
            pltpu.VMEM((_CHUNK, _D), jnp.float32),
            pltpu.SemaphoreType.DMA,
        ],
    )
    def body(mask_hbm, emb_hbm, out_hbm, midx_v, stage_v, gsem):
        cid = lax.axis_index("c")
        sid = lax.axis_index("s")
        wid = sid * _NC + cid
        base = wid * rpw

        pltpu.sync_copy(mask_hbm.at[pl.ds(base, rpw)], midx_v)

        for k in range(n_chunks):
            idx = midx_v.at[pl.ds(k * _CHUNK, _CHUNK)]
            pltpu.async_copy(emb_hbm.at[idx], stage_v, gsem).wait()
            pltpu.sync_copy(
                stage_v,
                out_hbm.at[pl.ds(base + k * _CHUNK, _CHUNK), pl.ds(_D, _D)])

    return body(mask_i32, emb_weight)


def _tc_body(mask_ref, seq_ref, buf_ref, out_ref):
    m = mask_ref[0]
    keep = (m.reshape(_ROWS, 1) == 0)
    out_ref[...] = jnp.where(keep, seq_ref[...], 0.0)


def kernel(seq, mask, emb_weight):
    B, S, D = seq.shape
    N = B * S
    G = N // _ROWS
    seq2 = seq.reshape(N, D)
    mask_i = mask.astype(jnp.int32)
    mask3 = mask_i.reshape(G, 1, _ROWS)

    buf = _sc_phase(mask_i.reshape(N), emb_weight, N)

    out = pl.pallas_call(
        _tc_body,
        grid=(G,),
        in_specs=[
            pl.BlockSpec((1, 1, _ROWS), lambda i: (i, 0, 0)),
            pl.BlockSpec((_ROWS, D), lambda i: (i, 0)),
            pl.BlockSpec((8, 128), lambda i: (0, 0)),
        ],
        out_specs=pl.BlockSpec((_ROWS, D), lambda i: (i, 0)),
        out_shape=jax.ShapeDtypeStruct((N, 2 * D), jnp.float32),
        input_output_aliases={2: 0},
    )(mask3, seq2, buf)
    return out.reshape(B, S, 2 * D)


# speedup vs baseline: 4.0044x; 4.0044x over previous
import functools

import jax
import jax.numpy as jnp
from jax import lax
from jax.experimental import pallas as pl
from jax.experimental.pallas import tpu as pltpu
from jax.experimental.pallas import tpu_sc as plsc

_D = 1024
_ROWS = 512
_NC = 2
_NS = 16
_NW = _NC * _NS
_CHUNK = 32
_REP = 512


def _sc_phase(gather_idx, emb_rep, n_rows):
    rpw = n_rows // _NW
    n_chunks = rpw // _CHUNK
    mesh = plsc.VectorSubcoreMesh(core_axis_name="c", subcore_axis_name="s")

    @functools.partial(
        pl.kernel,
        mesh=mesh,
        out_type=jax.ShapeDtypeStruct((n_rows, 2 * _D), jnp.float32),
        scratch_types=[
            pltpu.VMEM((rpw,), jnp.int32),
            pltpu.VMEM((2, _CHUNK, _D), jnp.float32),
            pltpu.SemaphoreType.DMA,
            pltpu.SemaphoreType.DMA,
        ],
    )
    def body(idx_hbm, table_hbm, out_hbm, midx_v, stage_v, sem0, sem1):
        cid = lax.axis_index("c")
        sid = lax.axis_index("s")
        wid = sid * _NC + cid
        base = wid * rpw

        pltpu.sync_copy(idx_hbm.at[pl.ds(base, rpw)], midx_v)

        sems = [sem0, sem1]
        copies = [None, None]

        def start(k):
            b = k % 2
            idx = midx_v.at[pl.ds(k * _CHUNK, _CHUNK)]
            copies[b] = pltpu.async_copy(
                table_hbm.at[idx], stage_v.at[b], sems[b])

        start(0)
        for k in range(n_chunks):
            b = k % 2
            copies[b].wait()
            if k + 1 < n_chunks:
                start(k + 1)
            pltpu.sync_copy(
                stage_v.at[b],
                out_hbm.at[pl.ds(base + k * _CHUNK, _CHUNK), pl.ds(_D, _D)])

    return body(gather_idx, emb_rep)


def _tc_body(mask_ref, seq_ref, buf_ref, out_ref):
    m = mask_ref[0]
    keep = (m.reshape(_ROWS, 1) == 0)
    out_ref[...] = jnp.where(keep, seq_ref[...], 0.0)


def kernel(seq, mask, emb_weight):
    B, S, D = seq.shape
    N = B * S
    G = N // _ROWS
    seq2 = seq.reshape(N, D)
    mask_i = mask.astype(jnp.int32)
    mask3 = mask_i.reshape(G, 1, _ROWS)

    emb_rep = jnp.tile(emb_weight, (_REP, 1))
    gather_idx = (jnp.arange(N, dtype=jnp.int32) % _REP) * 2 + mask_i.reshape(N)

    buf = _sc_phase(gather_idx, emb_rep, N)

    out = pl.pallas_call(
        _tc_body,
        grid=(G,),
        in_specs=[
            pl.BlockSpec((1, 1, _ROWS), lambda i: (i, 0, 0)),
            pl.BlockSpec((_ROWS, D), lambda i: (i, 0)),
            pl.BlockSpec((8, 128), lambda i: (0, 0)),
        ],
        out_specs=pl.BlockSpec((_ROWS, D), lambda i: (i, 0)),
        out_shape=jax.ShapeDtypeStruct((N, 2 * D), jnp.float32),
        input_output_aliases={2: 0},
    )(mask3, seq2, buf)
    return out.reshape(B, S, 2 * D)


# speedup vs baseline: 6.1312x; 1.5311x over previous
import functools

import jax
import jax.numpy as jnp
from jax import lax
from jax.experimental import pallas as pl
from jax.experimental.pallas import tpu as pltpu
from jax.experimental.pallas import tpu_sc as plsc

_D = 1024
_ROWS = 512
_NC = 2
_NS = 16
_NW = _NC * _NS
_CHUNK = 32
_REP = 512


def _sc_phase(mask_i, emb_weight, n_rows):
    rpw = n_rows // _NW
    n_groups = rpw // 16
    mesh = plsc.VectorSubcoreMesh(core_axis_name="c", subcore_axis_name="s")

    @functools.partial(
        pl.kernel,
        mesh=mesh,
        out_type=jax.ShapeDtypeStruct((n_rows, 2 * _D), jnp.float32),
        scratch_types=[
            pltpu.VMEM((rpw,), jnp.int32),
            pltpu.VMEM((2, _D), jnp.float32),
            pltpu.VMEM((_CHUNK, _D), jnp.float32),
            pltpu.SemaphoreType.DMA,
        ],
    )
    def body(mask_hbm, emb_hbm, out_hbm, midx_v, table_v, drain_v, wsem):
        cid = lax.axis_index("c")
        sid = lax.axis_index("s")
        wid = sid * _NC + cid
        base = wid * rpw

        pltpu.sync_copy(emb_hbm, table_v)
        pltpu.sync_copy(mask_hbm.at[pl.ds(base, rpw)], midx_v)

        lane = lax.iota(jnp.int32, 16)
        zero = jnp.zeros((16,), jnp.int32)

        def group(g, carry):
            v16 = midx_v[pl.ds(g * 16, 16)]
            for l in range(16):
                m_l = v16[l]

                row = base + g * 16 + l
                pltpu.async_copy(
                    table_v.at[pl.ds(m_l, 1)],
                    out_hbm.at[pl.ds(row, 1), pl.ds(_D, _D)],
                    wsem)
            return carry

        lax.fori_loop(0, n_groups, group, 0)

        def drain(k, carry):
            pltpu.make_async_copy(
                out_hbm.at[pl.ds(base, _CHUNK), pl.ds(_D, _D)],
                drain_v, wsem).wait()
            return carry

        lax.fori_loop(0, rpw // _CHUNK, drain, 0)

    return body(mask_i, emb_weight)


def _tc_body(mask_ref, seq_ref, buf_ref, out_ref):
    m = mask_ref[0]
    keep = (m.reshape(_ROWS, 1) == 0)
    out_ref[...] = jnp.where(keep, seq_ref[...], 0.0)


def kernel(seq, mask, emb_weight):
    B, S, D = seq.shape
    N = B * S
    G = N // _ROWS
    seq2 = seq.reshape(N, D)
    mask_i = mask.astype(jnp.int32)
    mask3 = mask_i.reshape(G, 1, _ROWS)

    buf = _sc_phase(mask_i.reshape(N), emb_weight, N)

    out = pl.pallas_call(
        _tc_body,
        grid=(G,),
        in_specs=[
            pl.BlockSpec((1, 1, _ROWS), lambda i: (i, 0, 0)),
            pl.BlockSpec((_ROWS, D), lambda i: (i, 0)),
            pl.BlockSpec((8, 128), lambda i: (0, 0)),
        ],
        out_specs=pl.BlockSpec((_ROWS, D), lambda i: (i, 0)),
        out_shape=jax.ShapeDtypeStruct((N, 2 * D), jnp.float32),
        input_output_aliases={2: 0},
    )(mask3, seq2, buf)
    return out.reshape(B, S, 2 * D)


# speedup vs baseline: 6.5002x; 1.0602x over previous
import functools

import jax
import jax.numpy as jnp
from jax import lax
from jax.experimental import pallas as pl
from jax.experimental.pallas import tpu as pltpu
from jax.experimental.pallas import tpu_sc as plsc

_D = 1024
_ROWS = 1024
_NC = 2
_NS = 16
_NW = _NC * _NS
_CHUNK = 32
_REP = 512


def _sc_phase(mask_i, emb_weight, n_rows):
    rpw = n_rows // _NW
    n_groups = rpw // 16
    mesh = plsc.VectorSubcoreMesh(core_axis_name="c", subcore_axis_name="s")

    @functools.partial(
        pl.kernel,
        mesh=mesh,
        out_type=jax.ShapeDtypeStruct((n_rows, 2 * _D), jnp.float32),
        scratch_types=[
            pltpu.VMEM((rpw,), jnp.int32),
            pltpu.VMEM((2, _D), jnp.float32),
            pltpu.VMEM((_CHUNK, _D), jnp.float32),
            pltpu.SemaphoreType.DMA,
        ],
    )
    def body(mask_hbm, emb_hbm, out_hbm, midx_v, table_v, drain_v, wsem):
        cid = lax.axis_index("c")
        sid = lax.axis_index("s")
        wid = sid * _NC + cid
        base = wid * rpw

        pltpu.sync_copy(emb_hbm, table_v)
        pltpu.sync_copy(mask_hbm.at[pl.ds(base, rpw)], midx_v)

        lane = lax.iota(jnp.int32, 16)
        zero = jnp.zeros((16,), jnp.int32)

        def group(g, carry):
            v16 = midx_v[pl.ds(g * 16, 16)]
            for l in range(16):
                m_l = v16[l]
                row = base + g * 16 + l
                pltpu.async_copy(
                    table_v.at[pl.ds(m_l, 1)],
                    out_hbm.at[pl.ds(row, 1), pl.ds(_D, _D)],
                    wsem)
            return carry

        lax.fori_loop(0, n_groups, group, 0)

        def drain(k, carry):
            pltpu.make_async_copy(
                out_hbm.at[pl.ds(base, _CHUNK), pl.ds(_D, _D)],
                drain_v, wsem).wait()
            return carry

        lax.fori_loop(0, rpw // _CHUNK, drain, 0)

    return body(mask_i, emb_weight)


def _tc_body(mask_ref, seq_ref, buf_ref, out_ref):
    m = mask_ref[0]
    keep = (m.reshape(_ROWS, 1) == 0)
    out_ref[...] = jnp.where(keep, seq_ref[...], 0.0)


def kernel(seq, mask, emb_weight):
    B, S, D = seq.shape
    N = B * S
    G = N // _ROWS
    seq2 = seq.reshape(N, D)
    mask_i = mask.astype(jnp.int32)
    mask3 = mask_i.reshape(G, 1, _ROWS)

    buf = _sc_phase(mask_i.reshape(N), emb_weight, N)

    out = pl.pallas_call(
        _tc_body,
        grid=(G,),
        in_specs=[
            pl.BlockSpec((1, 1, _ROWS), lambda i: (i, 0, 0)),
            pl.BlockSpec((_ROWS, D), lambda i: (i, 0)),
            pl.BlockSpec((8, 128), lambda i: (0, 0)),
        ],
        out_specs=pl.BlockSpec((_ROWS, D), lambda i: (i, 0)),
        out_shape=jax.ShapeDtypeStruct((N, 2 * D), jnp.float32),
        input_output_aliases={2: 0},
    )(mask3, seq2, buf)
    return out.reshape(B, S, 2 * D)


# speedup vs baseline: 6.5759x; 1.0117x over previous
import functools

import jax
import jax.numpy as jnp
from jax import lax
from jax.experimental import pallas as pl
from jax.experimental.pallas import tpu as pltpu
from jax.experimental.pallas import tpu_sc as plsc

_D = 1024
_ROWS = 2048
_NC = 2
_NS = 16
_NW = _NC * _NS
_CHUNK = 32
_REP = 512


def _sc_phase(mask_i, emb_weight, n_rows):
    rpw = n_rows // _NW
    n_groups = rpw // 16
    mesh = plsc.VectorSubcoreMesh(core_axis_name="c", subcore_axis_name="s")

    @functools.partial(
        pl.kernel,
        mesh=mesh,
        out_type=jax.ShapeDtypeStruct((n_rows, 2 * _D), jnp.float32),
        scratch_types=[
            pltpu.VMEM((rpw,), jnp.int32),
            pltpu.VMEM((2, _D), jnp.float32),
            pltpu.VMEM((_CHUNK, _D), jnp.float32),
            pltpu.SemaphoreType.DMA,
        ],
    )
    def body(mask_hbm, emb_hbm, out_hbm, midx_v, table_v, drain_v, wsem):
        cid = lax.axis_index("c")
        sid = lax.axis_index("s")
        wid = sid * _NC + cid
        base = wid * rpw

        pltpu.sync_copy(emb_hbm, table_v)
        pltpu.sync_copy(mask_hbm.at[pl.ds(base, rpw)], midx_v)

        lane = lax.iota(jnp.int32, 16)
        zero = jnp.zeros((16,), jnp.int32)

        def group(g, carry):
            v16 = midx_v[pl.ds(g * 16, 16)]
            for l in range(16):
                m_l = v16[l]
                row = base + g * 16 + l
                pltpu.async_copy(
                    table_v.at[pl.ds(m_l, 1)],
                    out_hbm.at[pl.ds(row, 1), pl.ds(_D, _D)],
                    wsem)
            return carry

        lax.fori_loop(0, n_groups, group, 0)

        def drain(k, carry):
            pltpu.make_async_copy(
                out_hbm.at[pl.ds(base, _CHUNK), pl.ds(_D, _D)],
                drain_v, wsem).wait()
            return carry

        lax.fori_loop(0, rpw // _CHUNK, drain, 0)

    return body(mask_i, emb_weight)


def _tc_body(mask_ref, seq_ref, buf_ref, out_ref):
    m = mask_ref[0]
    keep = (m.reshape(_ROWS, 1) == 0)
    out_ref[...] = jnp.where(keep, seq_ref[...], 0.0)


def kernel(seq, mask, emb_weight):
    B, S, D = seq.shape
    N = B * S
    G = N // _ROWS
    seq2 = seq.reshape(N, D)
    mask_i = mask.astype(jnp.int32)
    mask3 = mask_i.reshape(G, 1, _ROWS)

    buf = _sc_phase(mask_i.reshape(N), emb_weight, N)

    out = pl.pallas_call(
        _tc_body,
        grid=(G,),
        in_specs=[
            pl.BlockSpec((1, 1, _ROWS), lambda i: (i, 0, 0)),
            pl.BlockSpec((_ROWS, D), lambda i: (i, 0)),
            pl.BlockSpec((8, 128), lambda i: (0, 0)),
        ],
        out_specs=pl.BlockSpec((_ROWS, D), lambda i: (i, 0)),
        out_shape=jax.ShapeDtypeStruct((N, 2 * D), jnp.float32),
        input_output_aliases={2: 0},
    )(mask3, seq2, buf)
    return out.reshape(B, S, 2 * D)
